# Initial kernel scaffold; baseline (speedup 1.0000x reference)
import jax
import jax.numpy as jnp
from jax.experimental import pallas as pl


def kernel(x, edge_index_1, edge_index_2, params):
    raise NotImplementedError("write your pallas kernel here")


# speedup vs baseline: 1.3951x; 1.3951x over previous
import functools

import jax
import jax.numpy as jnp
from jax import lax
from jax.experimental import pallas as pl
from jax.experimental.pallas import tpu as pltpu

_EPS = 1e-5
_PREC = lax.Precision.HIGHEST


def _dot(a, b):
    return jnp.matmul(a, b, precision=_PREC)


def _edge_mm_body(h1_ref, w_ref, b_ref, sg_ref, h2_ref, s_ref, q_ref):
    h1 = h1_ref[...]
    h2 = jnp.maximum(_dot(h1, w_ref[...]) + b_ref[...], 0.0)
    h2_ref[...] = h2 * sg_ref[...]

    @pl.when(pl.program_id(0) == 0)
    def _init():
        s_ref[...] = jnp.zeros_like(s_ref)
        q_ref[...] = jnp.zeros_like(q_ref)

    s_ref[...] += jnp.sum(h2, axis=0, keepdims=True)
    q_ref[...] += jnp.sum(h2 * h2, axis=0, keepdims=True)


def _edge_mm(h1, w, b, sign, blk=2560):
    e, f = h1.shape
    grid = e // blk
    h2s, s, q = pl.pallas_call(
        _edge_mm_body,
        grid=(grid,),
        in_specs=[
            pl.BlockSpec((blk, f), lambda i: (i, 0)),
            pl.BlockSpec((f, f), lambda i: (0, 0)),
            pl.BlockSpec((1, f), lambda i: (0, 0)),
            pl.BlockSpec((1, f), lambda i: (0, 0)),
        ],
        out_specs=[
            pl.BlockSpec((blk, f), lambda i: (i, 0)),
            pl.BlockSpec((1, f), lambda i: (0, 0)),
            pl.BlockSpec((1, f), lambda i: (0, 0)),
        ],
        out_shape=[
            jax.ShapeDtypeStruct((e, f), jnp.float32),
            jax.ShapeDtypeStruct((1, f), jnp.float32),
            jax.ShapeDtypeStruct((1, f), jnp.float32),
        ],
    )(h1, w, b.reshape(1, f), sign.reshape(1, f))
    return h2s, s[0], q[0]


def _mm_body(x_ref, w_ref, b_ref, o_ref):
    o_ref[...] = jnp.maximum(_dot(x_ref[...], w_ref[...]) + b_ref[...], 0.0)


def _mm_relu(x, w, b, blk=2000):
    n, k = x.shape
    f = w.shape[1]
    return pl.pallas_call(
        _mm_body,
        grid=(n // blk,),
        in_specs=[
            pl.BlockSpec((blk, k), lambda i: (i, 0)),
            pl.BlockSpec((k, f), lambda i: (0, 0)),
            pl.BlockSpec((1, f), lambda i: (0, 0)),
        ],
        out_specs=pl.BlockSpec((blk, f), lambda i: (i, 0)),
        out_shape=jax.ShapeDtypeStruct((n, f), jnp.float32),
    )(x, w, b.reshape(1, f))


def _bn_affine(s, q, cnt, gamma, beta):
    mean = s / cnt
    var = q / cnt - mean * mean
    a = gamma * lax.rsqrt(jnp.maximum(var, 0.0) + _EPS)
    c = beta - mean * a
    return a, c


def _one_conv(x, edge_index, layers):
    n, d = x.shape
    e = edge_index.shape[1]
    src, dst = edge_index[0], edge_index[1]
    mask = src != dst
    nvalid = jnp.sum(mask.astype(jnp.float32))
    cnt = nvalid + n

    w1, b1 = layers[0]["W"], layers[0]["b"]
    g1, be1 = layers[0]["gamma"], layers[0]["beta"]
    w2, b2 = layers[1]["W"], layers[1]["b"]
    g2, be2 = layers[1]["gamma"], layers[1]["beta"]
    f = w1.shape[1]

    pd = _dot(x, w1[:d] - w1[d:]) + b1
    ps = _dot(x, w1[d:])

    h1 = jnp.maximum(pd[dst] + ps[src], 0.0) * mask[:, None].astype(jnp.float32)
    h1_self = jnp.maximum(pd + ps, 0.0)

    s1 = jnp.sum(h1, axis=0) + jnp.sum(h1_self, axis=0)
    q1 = jnp.sum(h1 * h1, axis=0) + jnp.sum(h1_self * h1_self, axis=0)
    a1, c1 = _bn_affine(s1, q1, cnt, g1, be1)

    w2p = a1[:, None] * w2
    b2p = _dot(c1, w2) + b2
    sign2 = jnp.sign(g2)

    h2s, s2, q2 = _edge_mm(h1, w2p, b2p, sign2)
    h2_self = jnp.maximum(_dot(h1_self, w2p) + b2p, 0.0)

    r0 = jnp.maximum(b2p, 0.0)
    nmask = jnp.float32(e) - nvalid
    s2 = sign2 * s2 + jnp.sum(h2_self, axis=0) - nmask * r0
    q2 = q2 + jnp.sum(h2_self * h2_self, axis=0) - nmask * r0 * r0
    a2, c2 = _bn_affine(s2, q2, cnt, g2, be2)

    h2s_masked = jnp.where(mask[:, None], h2s, -jnp.inf)
    acc = jax.ops.segment_max(h2s_masked, dst, num_segments=n)
    acc = jnp.maximum(acc, h2_self * sign2)
    return acc * jnp.abs(a2) + c2


def kernel(x, edge_index_1, edge_index_2, params):
    n = x.shape[0]
    o1 = _one_conv(x, edge_index_1, params["conv1"])
    o2 = _one_conv(x, edge_index_2, params["conv2"])
    out = jnp.concatenate([o1, o2], axis=1)

    p = params["mlp"][0]
    h = _mm_relu(out, p["W"], p["b"])
    s = jnp.sum(h, axis=0)
    q = jnp.sum(h * h, axis=0)
    a, c = _bn_affine(s, q, jnp.float32(n), p["gamma"], p["beta"])
    return h * a + c


# speedup vs baseline: 2.6626x; 1.9086x over previous
import functools

import jax
import jax.numpy as jnp
from jax import lax
from jax.experimental import pallas as pl
from jax.experimental.pallas import tpu as pltpu
from jax.experimental.pallas import tpu_sc as plsc

_NC = 2
_NS = 16
_NW = _NC * _NS
_LANES = 16

_EPS = 1e-5
_PREC = lax.Precision.HIGHEST


def _dot(a, b):
    return jnp.matmul(a, b, precision=_PREC)


def _gather_body(npad, chunk, pd_hbm, ps_hbm, src_hbm, dst_hbm,
                 h1_hbm, stats_hbm, srcv, dstv, idxd, idxs, bufd, bufs,
                 statv, sem):
    e_total = src_hbm.shape[0]
    per_w = e_total // _NW
    nchunks = per_w // chunk
    wid = lax.axis_index("s") * _NC + lax.axis_index("c")
    base_w = wid * per_w

    dummy = jnp.full((_LANES,), npad, jnp.int32)

    def chunk_body(ci, acc):
        base = base_w + ci * chunk
        pltpu.sync_copy(src_hbm.at[pl.ds(base, chunk)], srcv)
        pltpu.sync_copy(dst_hbm.at[pl.ds(base, chunk)], dstv)

        def fix(j, _):
            sl = pl.ds(j * _LANES, _LANES)
            s = srcv[sl]
            d = dstv[sl]
            m = s == d
            idxd[sl] = jnp.where(m, dummy, d)
            idxs[sl] = jnp.where(m, dummy, s)
            return 0

        lax.fori_loop(0, chunk // _LANES, fix, 0)
        cp1 = pltpu.async_copy(pd_hbm.at[idxd], bufd, sem)
        cp2 = pltpu.async_copy(ps_hbm.at[idxs], bufs, sem)
        cp1.wait()
        cp2.wait()

        def body(i, carry):
            out = list(carry)
            for c in range(4):
                sl = pl.ds(c * _LANES, _LANES)
                h = jnp.maximum(bufd[i, sl] + bufs[i, sl], 0.0)
                bufd[i, sl] = h
                out[c] = out[c] + h
                out[4 + c] = out[4 + c] + h * h
            return tuple(out)

        acc = lax.fori_loop(0, chunk, body, acc)
        pltpu.sync_copy(bufd, h1_hbm.at[pl.ds(base, chunk)])
        return acc

    zero = jnp.zeros((_LANES,), jnp.float32)
    acc = lax.fori_loop(0, nchunks, chunk_body, (zero,) * 8)
    for c in range(8):
        statv[pl.ds(c * _LANES, _LANES)] = acc[c]
    pltpu.sync_copy(statv, stats_hbm.at[wid])


def _sc_gather(pd_ext, ps_ext, src, dst, chunk=400):
    e_total = src.shape[0]
    npad = pd_ext.shape[0] - 8
    f = pd_ext.shape[1]
    mesh = plsc.VectorSubcoreMesh(core_axis_name="c", subcore_axis_name="s")
    kern = functools.partial(
        pl.kernel,
        out_type=[
            jax.ShapeDtypeStruct((e_total, f), jnp.float32),
            jax.ShapeDtypeStruct((_NW, 2 * f), jnp.float32),
        ],
        mesh=mesh,
        scratch_types=[
            pltpu.VMEM((chunk,), jnp.int32),
            pltpu.VMEM((chunk,), jnp.int32),
            pltpu.VMEM((chunk,), jnp.int32),
            pltpu.VMEM((chunk,), jnp.int32),
            pltpu.VMEM((chunk, f), jnp.float32),
            pltpu.VMEM((chunk, f), jnp.float32),
            pltpu.VMEM((2 * f,), jnp.float32),
            pltpu.SemaphoreType.DMA,
        ],
        compiler_params=pltpu.CompilerParams(use_tc_tiling_on_sc=False),
    )(functools.partial(_gather_body, npad, chunk))
    h1, stats = kern(pd_ext, ps_ext, src, dst)
    return h1, stats


_EG = 4
_FG = 8
_FPT = 8


def _scatter_body(npad, chunk, h2t_hbm, src_hbm, dst_hbm, acc_hbm,
                  srcv, dstv, idxv, vbuf, accv, scratch, sem):
    e_total = src_hbm.shape[0]
    per_g = e_total // _EG
    nchunks = per_g // chunk
    wid = lax.axis_index("s") * _NC + lax.axis_index("c")
    g = wid // _FG
    fg = wid % _FG
    base_g = g * per_g

    neg = jnp.full((_LANES,), -3.0e38, jnp.float32)
    nacc = npad * _FPT

    def init_body(i, _):
        accv[pl.ds(i * _LANES, _LANES)] = neg
        return 0

    lax.fori_loop(0, nacc // _LANES, init_body, 0)

    iota = lax.iota(jnp.int32, _LANES)
    dummy = jnp.full((_LANES,), npad - 1, jnp.int32)

    def chunk_loop(ci, _):
        base = base_g + ci * chunk
        pltpu.sync_copy(src_hbm.at[pl.ds(base, chunk)], srcv)
        pltpu.sync_copy(dst_hbm.at[pl.ds(base, chunk)], dstv)

        def fix(j, _):
            sl = pl.ds(j * _LANES, _LANES)
            s = srcv[sl]
            d = dstv[sl]
            idxv[sl] = jnp.where(s == d, dummy, d)
            return 0

        lax.fori_loop(0, chunk // _LANES, fix, 0)
        cps = [
            pltpu.async_copy(
                h2t_hbm.at[fg * _FPT + k, pl.ds(base, chunk)],
                vbuf.at[k], sem)
            for k in range(_FPT)
        ]
        for cp in cps:
            cp.wait()

        def group(j, _):
            sl = pl.ds(j * _LANES, _LANES)
            idx = idxv[sl]
            idx8 = idx * _FPT
            nrounds = 2

            def step(r, carry):
                rem = carry[0] != 0
                plsc.store_scatter(scratch, [idx], iota, mask=rem)
                back = plsc.load_gather(scratch, [idx])
                win = rem & (back == iota)
                for k in range(_FPT):
                    v = vbuf[k, sl]
                    cur = plsc.load_gather(accv, [idx8 + k])
                    plsc.store_scatter(accv, [idx8 + k],
                                       jnp.maximum(cur, v), mask=win)
                rem2 = rem & jnp.logical_not(win)
                return (jnp.where(rem2, 1, 0).astype(jnp.int32),)

            lax.fori_loop(0, nrounds, step,
                          (jnp.ones((_LANES,), jnp.int32),))
            return 0

        lax.fori_loop(0, chunk // _LANES, group, 0)
        return 0

    lax.fori_loop(0, nchunks, chunk_loop, 0)
    pltpu.sync_copy(accv, acc_hbm.at[g, pl.ds(fg * _FPT * npad, nacc)])


def _sc_scatter_max(h2t, src, dst, npad):
    e_total = src.shape[0]
    chunk = 2000
    mesh = plsc.VectorSubcoreMesh(core_axis_name="c", subcore_axis_name="s")
    kern = functools.partial(
        pl.kernel,
        out_type=jax.ShapeDtypeStruct((_EG, _FG * _FPT * npad), jnp.float32),
        mesh=mesh,
        scratch_types=[
            pltpu.VMEM((chunk,), jnp.int32),
            pltpu.VMEM((chunk,), jnp.int32),
            pltpu.VMEM((chunk,), jnp.int32),
            pltpu.VMEM((_FPT, chunk), jnp.float32),
            pltpu.VMEM((npad * _FPT,), jnp.float32),
            pltpu.VMEM((npad,), jnp.int32),
            pltpu.SemaphoreType.DMA,
        ],
        compiler_params=pltpu.CompilerParams(use_tc_tiling_on_sc=False),
    )(functools.partial(_scatter_body, npad, chunk))
    return kern(h2t, src, dst)


def _edge_mm_body(h1_ref, w_ref, b_ref, sg_ref, h2_ref, s_ref, q_ref):
    h1 = h1_ref[...]
    h2 = jnp.maximum(_dot(h1, w_ref[...]) + b_ref[...], 0.0)
    h2_ref[...] = (h2 * sg_ref[...]).T

    @pl.when(pl.program_id(0) == 0)
    def _init():
        s_ref[...] = jnp.zeros_like(s_ref)
        q_ref[...] = jnp.zeros_like(q_ref)

    s_ref[...] += jnp.sum(h2, axis=0, keepdims=True)
    q_ref[...] += jnp.sum(h2 * h2, axis=0, keepdims=True)


def _edge_mm(h1, w, b, sign, blk=2560):
    e, f = h1.shape
    grid = e // blk
    h2s, s, q = pl.pallas_call(
        _edge_mm_body,
        grid=(grid,),
        in_specs=[
            pl.BlockSpec((blk, f), lambda i: (i, 0)),
            pl.BlockSpec((f, f), lambda i: (0, 0)),
            pl.BlockSpec((1, f), lambda i: (0, 0)),
            pl.BlockSpec((1, f), lambda i: (0, 0)),
        ],
        out_specs=[
            pl.BlockSpec((f, blk), lambda i: (0, i)),
            pl.BlockSpec((1, f), lambda i: (0, 0)),
            pl.BlockSpec((1, f), lambda i: (0, 0)),
        ],
        out_shape=[
            jax.ShapeDtypeStruct((f, e), jnp.float32),
            jax.ShapeDtypeStruct((1, f), jnp.float32),
            jax.ShapeDtypeStruct((1, f), jnp.float32),
        ],
    )(h1, w, b.reshape(1, f), sign.reshape(1, f))
    return h2s, s[0], q[0]


def _mm_body(x_ref, w_ref, b_ref, o_ref):
    o_ref[...] = jnp.maximum(_dot(x_ref[...], w_ref[...]) + b_ref[...], 0.0)


def _mm_relu(x, w, b, blk=2000):
    n, k = x.shape
    f = w.shape[1]
    return pl.pallas_call(
        _mm_body,
        grid=(n // blk,),
        in_specs=[
            pl.BlockSpec((blk, k), lambda i: (i, 0)),
            pl.BlockSpec((k, f), lambda i: (0, 0)),
            pl.BlockSpec((1, f), lambda i: (0, 0)),
        ],
        out_specs=pl.BlockSpec((blk, f), lambda i: (i, 0)),
        out_shape=jax.ShapeDtypeStruct((n, f), jnp.float32),
    )(x, w, b.reshape(1, f))


def _bn_affine(s, q, cnt, gamma, beta):
    mean = s / cnt
    var = q / cnt - mean * mean
    a = gamma * lax.rsqrt(jnp.maximum(var, 0.0) + _EPS)
    c = beta - mean * a
    return a, c


def _one_conv(x, edge_index, layers):
    n, d = x.shape
    e = edge_index.shape[1]
    src, dst = edge_index[0], edge_index[1]
    mask = src != dst
    nvalid = jnp.sum(mask.astype(jnp.float32))
    cnt = nvalid + n

    w1, b1 = layers[0]["W"], layers[0]["b"]
    g1, be1 = layers[0]["gamma"], layers[0]["beta"]
    w2, b2 = layers[1]["W"], layers[1]["b"]
    g2, be2 = layers[1]["gamma"], layers[1]["beta"]
    f = w1.shape[1]

    pad = jnp.zeros((8, f), jnp.float32)
    pd = jnp.concatenate([_dot(x, w1[:d] - w1[d:]) + b1, pad], axis=0)
    ps = jnp.concatenate([_dot(x, w1[d:]), pad], axis=0)

    h1, stats = _sc_gather(pd, ps, src, dst)
    h1_self = jnp.maximum(pd[:n] + ps[:n], 0.0)

    s1 = jnp.sum(stats[:, :f], axis=0) + jnp.sum(h1_self, axis=0)
    q1 = jnp.sum(stats[:, f:], axis=0) + jnp.sum(h1_self * h1_self, axis=0)
    a1, c1 = _bn_affine(s1, q1, cnt, g1, be1)

    w2p = a1[:, None] * w2
    b2p = _dot(c1, w2) + b2
    sign2 = jnp.sign(g2)

    h2t, s2, q2 = _edge_mm(h1, w2p, b2p, sign2)
    h2_self = jnp.maximum(_dot(h1_self, w2p) + b2p, 0.0)

    r0 = jnp.maximum(b2p, 0.0)
    nmask = jnp.float32(e) - nvalid
    s2 = sign2 * s2 + jnp.sum(h2_self, axis=0) - nmask * r0
    q2 = q2 + jnp.sum(h2_self * h2_self, axis=0) - nmask * r0 * r0
    a2, c2 = _bn_affine(s2, q2, cnt, g2, be2)

    h2s_masked = jnp.where(mask[:, None], h2t.T, -jnp.inf)
    acc = jax.ops.segment_max(h2s_masked, dst, num_segments=n)
    acc = jnp.maximum(acc, h2_self * sign2)
    return acc * jnp.abs(a2) + c2


def kernel(x, edge_index_1, edge_index_2, params):
    n = x.shape[0]
    o1 = _one_conv(x, edge_index_1, params["conv1"])
    o2 = _one_conv(x, edge_index_2, params["conv2"])
    out = jnp.concatenate([o1, o2], axis=1)

    p = params["mlp"][0]
    h = _mm_relu(out, p["W"], p["b"])
    s = jnp.sum(h, axis=0)
    q = jnp.sum(h * h, axis=0)
    a, c = _bn_affine(s, q, jnp.float32(n), p["gamma"], p["beta"])
    return h * a + c
